# Initial kernel scaffold; baseline (speedup 1.0000x reference)
import jax
import jax.numpy as jnp
from jax.experimental import pallas as pl


def kernel(mh_output, W, b):
    raise NotImplementedError("write your pallas kernel here")


# speedup vs baseline: 5.0048x; 5.0048x over previous
import functools

import jax
import jax.numpy as jnp
from jax.experimental import pallas as pl
from jax.experimental.pallas import tpu as pltpu

_NUM_EXPERTS = 64
_TOP_K = 8
_TB = 512


def _router_block(x_ref, w_ref, b_ref, out_ref, idx_ref):
    x = x_ref[...]
    w = w_ref[...]
    logits = jax.lax.dot_general(
        x, w, (((1,), (1,)), ((), ())), preferred_element_type=jnp.float32
    )
    logits = logits + b_ref[...]

    tb = logits.shape[0]
    iota = jax.lax.broadcasted_iota(jnp.int32, (tb, _NUM_EXPERTS), 1)
    work = logits
    mask = jnp.zeros((tb, _NUM_EXPERTS), dtype=jnp.bool_)
    idx_cols = []
    top_val = None
    neg_inf = jnp.float32(-jnp.inf)
    for k in range(_TOP_K):
        m = jnp.max(work, axis=1, keepdims=True)
        if k == 0:
            top_val = m
        idx = jnp.min(
            jnp.where(work == m, iota, _NUM_EXPERTS), axis=1, keepdims=True
        )
        hit = iota == idx
        mask = jnp.logical_or(mask, hit)
        work = jnp.where(hit, neg_inf, work)
        idx_cols.append(idx)

    e = jnp.where(mask, jnp.exp(logits - top_val), 0.0)
    denom = jnp.sum(e, axis=1, keepdims=True)
    out_ref[...] = e / denom
    idx_ref[...] = jnp.concatenate(idx_cols, axis=1)


@jax.jit
def kernel(mh_output, W, b):
    B, S, E = mh_output.shape
    n_tok = B * S
    x = mh_output.reshape(n_tok, E)
    grid = (n_tok // _TB,)
    router, idx = pl.pallas_call(
        _router_block,
        grid=grid,
        in_specs=[
            pl.BlockSpec((_TB, E), lambda i: (i, 0)),
            pl.BlockSpec((_NUM_EXPERTS, E), lambda i: (0, 0)),
            pl.BlockSpec((_NUM_EXPERTS,), lambda i: (0,)),
        ],
        out_specs=[
            pl.BlockSpec((_TB, _NUM_EXPERTS), lambda i: (i, 0)),
            pl.BlockSpec((_TB, _TOP_K), lambda i: (i, 0)),
        ],
        out_shape=[
            jax.ShapeDtypeStruct((n_tok, _NUM_EXPERTS), jnp.float32),
            jax.ShapeDtypeStruct((n_tok, _TOP_K), jnp.int32),
        ],
    )(x, W, b)
    return router.reshape(B, S, _NUM_EXPERTS), idx.reshape(B, S, _TOP_K)


# speedup vs baseline: 6.0759x; 1.2140x over previous
import jax
import jax.numpy as jnp
from jax.experimental import pallas as pl

_NUM_EXPERTS = 64
_TOP_K = 8
_TB = 512


def _router_block(x_ref, w_ref, b_ref, out_ref, idx_ref):
    x = x_ref[...]
    w = w_ref[...]
    logits = jax.lax.dot_general(
        w, x, (((1,), (1,)), ((), ())), preferred_element_type=jnp.float32
    )
    logits = logits + b_ref[...]

    tb = logits.shape[1]
    fiota = jax.lax.broadcasted_iota(jnp.int32, (_NUM_EXPERTS, tb), 0).astype(
        jnp.float32
    )
    work = logits
    idx_rows = []
    top_val = None
    neg_inf = jnp.float32(-jnp.inf)
    for k in range(_TOP_K):
        m = jnp.max(work, axis=0, keepdims=True)
        if k == 0:
            top_val = m
        idx = jnp.min(
            jnp.where(work == m, fiota, jnp.float32(_NUM_EXPERTS)),
            axis=0,
            keepdims=True,
        )
        work = jnp.where(fiota == idx, neg_inf, work)
        idx_rows.append(idx)

    e = jnp.where(work == neg_inf, jnp.exp(logits - top_val), 0.0)
    denom = jnp.sum(e, axis=0, keepdims=True)
    out_ref[...] = (e / denom).T
    idxs = jnp.concatenate(idx_rows, axis=0)
    idx_ref[...] = idxs.T.astype(jnp.int32)


@jax.jit
def kernel(mh_output, W, b):
    B, S, E = mh_output.shape
    n_tok = B * S
    x = mh_output.reshape(n_tok, E)
    grid = (n_tok // _TB,)
    router, idx = pl.pallas_call(
        _router_block,
        grid=grid,
        in_specs=[
            pl.BlockSpec((_TB, E), lambda i: (i, 0)),
            pl.BlockSpec((_NUM_EXPERTS, E), lambda i: (0, 0)),
            pl.BlockSpec((_NUM_EXPERTS, 1), lambda i: (0, 0)),
        ],
        out_specs=[
            pl.BlockSpec((_TB, _NUM_EXPERTS), lambda i: (i, 0)),
            pl.BlockSpec((_TB, _TOP_K), lambda i: (i, 0)),
        ],
        out_shape=[
            jax.ShapeDtypeStruct((n_tok, _NUM_EXPERTS), jnp.float32),
            jax.ShapeDtypeStruct((n_tok, _TOP_K), jnp.int32),
        ],
    )(x, W, b.reshape(_NUM_EXPERTS, 1))
    return router.reshape(B, S, _NUM_EXPERTS), idx.reshape(B, S, _TOP_K)


# speedup vs baseline: 6.4598x; 1.0632x over previous
import jax
import jax.numpy as jnp
from jax.experimental import pallas as pl

_NUM_EXPERTS = 64
_TOP_K = 8
_TB = 1024


def _router_block(x_ref, w_ref, b_ref, out_ref, idx_ref):
    x = x_ref[...]
    w = w_ref[...]
    logits = jax.lax.dot_general(
        w, x, (((1,), (1,)), ((), ())), preferred_element_type=jnp.float32
    )
    logits = logits + b_ref[...]

    tb = logits.shape[1]
    fiota = jax.lax.broadcasted_iota(jnp.int32, (_NUM_EXPERTS, tb), 0).astype(
        jnp.float32
    )
    work = logits
    idx_rows = []
    top_val = None
    neg_inf = jnp.float32(-jnp.inf)
    for k in range(_TOP_K):
        m = jnp.max(work, axis=0, keepdims=True)
        if k == 0:
            top_val = m
        idx = jnp.min(
            jnp.where(work == m, fiota, jnp.float32(_NUM_EXPERTS)),
            axis=0,
            keepdims=True,
        )
        work = jnp.where(fiota == idx, neg_inf, work)
        idx_rows.append(idx)

    e = jnp.where(work == neg_inf, jnp.exp(logits - top_val), 0.0)
    denom = jnp.sum(e, axis=0, keepdims=True)
    out_ref[...] = (e / denom).T
    idxs = jnp.concatenate(idx_rows, axis=0)
    idx_ref[...] = idxs.T.astype(jnp.int32)


@jax.jit
def kernel(mh_output, W, b):
    B, S, E = mh_output.shape
    n_tok = B * S
    x = mh_output.reshape(n_tok, E)
    grid = (n_tok // _TB,)
    router, idx = pl.pallas_call(
        _router_block,
        grid=grid,
        in_specs=[
            pl.BlockSpec((_TB, E), lambda i: (i, 0)),
            pl.BlockSpec((_NUM_EXPERTS, E), lambda i: (0, 0)),
            pl.BlockSpec((_NUM_EXPERTS, 1), lambda i: (0, 0)),
        ],
        out_specs=[
            pl.BlockSpec((_TB, _NUM_EXPERTS), lambda i: (i, 0)),
            pl.BlockSpec((_TB, _TOP_K), lambda i: (i, 0)),
        ],
        out_shape=[
            jax.ShapeDtypeStruct((n_tok, _NUM_EXPERTS), jnp.float32),
            jax.ShapeDtypeStruct((n_tok, _TOP_K), jnp.int32),
        ],
    )(x, W, b.reshape(_NUM_EXPERTS, 1))
    return router.reshape(B, S, _NUM_EXPERTS), idx.reshape(B, S, _TOP_K)


# speedup vs baseline: 6.4717x; 1.0018x over previous
import jax
import jax.numpy as jnp
from jax.experimental import pallas as pl

_NUM_EXPERTS = 64
_TOP_K = 8
_TB = 1024
_KH = 2048


def _router_block(x1_ref, x2_ref, w_ref, b_ref, out_ref, idx_ref):
    w = w_ref[...]
    dn = (((1,), (1,)), ((), ()))
    logits = jax.lax.dot_general(
        w[:, :_KH], x1_ref[...], dn, preferred_element_type=jnp.float32
    )
    logits = logits + jax.lax.dot_general(
        w[:, _KH:], x2_ref[...], dn, preferred_element_type=jnp.float32
    )
    logits = logits + b_ref[...]

    tb = logits.shape[1]
    fiota = jax.lax.broadcasted_iota(jnp.int32, (_NUM_EXPERTS, tb), 0).astype(
        jnp.float32
    )
    work = logits
    idx_rows = []
    top_val = None
    neg_inf = jnp.float32(-jnp.inf)
    for k in range(_TOP_K):
        m = jnp.max(work, axis=0, keepdims=True)
        if k == 0:
            top_val = m
        idx = jnp.min(
            jnp.where(work == m, fiota, jnp.float32(_NUM_EXPERTS)),
            axis=0,
            keepdims=True,
        )
        work = jnp.where(fiota == idx, neg_inf, work)
        idx_rows.append(idx)

    e = jnp.where(work == neg_inf, jnp.exp(logits - top_val), 0.0)
    denom = jnp.sum(e, axis=0, keepdims=True)
    out_ref[...] = (e / denom).T
    idxs = jnp.concatenate(idx_rows, axis=0)
    idx_ref[...] = idxs.T.astype(jnp.int32)


@jax.jit
def kernel(mh_output, W, b):
    B, S, E = mh_output.shape
    n_tok = B * S
    x = mh_output.reshape(n_tok, E)
    grid = (n_tok // _TB,)
    router, idx = pl.pallas_call(
        _router_block,
        grid=grid,
        in_specs=[
            pl.BlockSpec((_TB, _KH), lambda i: (i, 0)),
            pl.BlockSpec((_TB, _KH), lambda i: (i, 1)),
            pl.BlockSpec((_NUM_EXPERTS, E), lambda i: (0, 0)),
            pl.BlockSpec((_NUM_EXPERTS, 1), lambda i: (0, 0)),
        ],
        out_specs=[
            pl.BlockSpec((_TB, _NUM_EXPERTS), lambda i: (i, 0)),
            pl.BlockSpec((_TB, _TOP_K), lambda i: (i, 0)),
        ],
        out_shape=[
            jax.ShapeDtypeStruct((n_tok, _NUM_EXPERTS), jnp.float32),
            jax.ShapeDtypeStruct((n_tok, _TOP_K), jnp.int32),
        ],
    )(x, x, W, b.reshape(_NUM_EXPERTS, 1))
    return router.reshape(B, S, _NUM_EXPERTS), idx.reshape(B, S, _TOP_K)
